# baseline (speedup 1.0000x reference)
import functools

import jax
import jax.numpy as jnp
from jax import lax
from jax.experimental import pallas as pl
from jax.experimental.pallas import tpu as pltpu
from jax.experimental.pallas import tpu_sc as plsc

BATCH = 16384
EMB = 16
NC = 2
NS = 16
NW = NC * NS
BPW = BATCH // NW
L = 16


def _mesh():
    return plsc.VectorSubcoreMesh(core_axis_name="c", subcore_axis_name="s")


@functools.partial(
    pl.kernel,
    out_type=(
        jax.ShapeDtypeStruct((NW, EMB), jnp.float32),
        jax.ShapeDtypeStruct((BATCH,), jnp.float32),
    ),
    mesh=_mesh(),
    compiler_params=pltpu.CompilerParams(use_tc_tiling_on_sc=False, needs_layout_passes=False),
    scratch_types=[
        pltpu.VMEM((BPW,), jnp.int32),
        pltpu.VMEM((BPW,), jnp.int32),
        pltpu.VMEM((BPW, EMB), jnp.float32),
        pltpu.VMEM((BPW, EMB), jnp.float32),
        pltpu.VMEM((BPW,), jnp.float32),
        pltpu.VMEM((BPW,), jnp.float32),
        pltpu.VMEM((BPW,), jnp.float32),
        pltpu.VMEM((EMB,), jnp.float32),
        pltpu.SemaphoreType.DMA,
        pltpu.SemaphoreType.DMA,
        pltpu.SemaphoreType.DMA,
        pltpu.SemaphoreType.DMA,
    ],
)
def _gather_partials(uidx_hbm, midx_hbm, uemb_hbm, memb_hbm, ub_hbm, mb_hbm,
                     part_hbm, s_hbm,
                     uidx_v, midx_v, urows_v, mrows_v, ub_v, mb_v, s_v,
                     acc_v, sem0, sem1, sem2, sem3):
    wid = lax.axis_index("s") * NC + lax.axis_index("c")
    base = wid * BPW
    pltpu.sync_copy(uidx_hbm.at[pl.ds(base, BPW)], uidx_v)
    pltpu.sync_copy(midx_hbm.at[pl.ds(base, BPW)], midx_v)
    c0 = pltpu.async_copy(uemb_hbm.at[uidx_v], urows_v, sem0)
    c1 = pltpu.async_copy(memb_hbm.at[midx_v], mrows_v, sem1)
    c2 = pltpu.async_copy(ub_hbm.at[uidx_v], ub_v, sem2)
    c3 = pltpu.async_copy(mb_hbm.at[midx_v], mb_v, sem3)
    c2.wait()
    c3.wait()
    for i in range(BPW // L):
        s_v[pl.ds(i * L, L)] = ub_v[pl.ds(i * L, L)] + mb_v[pl.ds(i * L, L)]
    c0.wait()
    c1.wait()

    def body(i, acc):
        return acc + urows_v[i, :] * mrows_v[i, :]

    acc = lax.fori_loop(0, BPW, body, jnp.zeros((EMB,), jnp.float32))
    acc_v[...] = acc
    pltpu.sync_copy(s_v, s_hbm.at[pl.ds(base, BPW)])
    pltpu.sync_copy(acc_v, part_hbm.at[wid])


@functools.partial(
    pl.kernel,
    out_type=jax.ShapeDtypeStruct((BATCH,), jnp.float32),
    mesh=_mesh(),
    compiler_params=pltpu.CompilerParams(use_tc_tiling_on_sc=False, needs_layout_passes=False),
    scratch_types=[
        pltpu.VMEM((NW, EMB), jnp.float32),
        pltpu.VMEM((BPW,), jnp.float32),
        pltpu.VMEM((BPW,), jnp.float32),
    ],
)
def _reduce_sigmoid(part_hbm, s_hbm, out_hbm, part_v, s_v, out_v):
    wid = lax.axis_index("s") * NC + lax.axis_index("c")
    base = wid * BPW
    pltpu.sync_copy(part_hbm, part_v)
    pltpu.sync_copy(s_hbm.at[pl.ds(base, BPW)], s_v)
    acc = part_v[0, :]
    for j in range(1, NW):
        acc = acc + part_v[j, :]
    total = lax.reduce_sum_p.bind(acc, axes=(0,))
    for i in range(BPW // L):
        x = s_v[pl.ds(i * L, L)] + total
        out_v[pl.ds(i * L, L)] = 1.0 / (1.0 + jnp.exp(-x))
    pltpu.sync_copy(out_v, out_hbm.at[pl.ds(base, BPW)])


def kernel(inputs, user_embedding, user_bias, movie_embedding, movie_bias):
    uidx = inputs[:, 0]
    midx = inputs[:, 1]
    part, s = _gather_partials(uidx, midx, user_embedding, movie_embedding,
                               user_bias.reshape(-1), movie_bias.reshape(-1))
    out = _reduce_sigmoid(part, s)
    return out.reshape(BATCH, 1)
